# baseline (speedup 1.0000x reference)
import functools

import jax
import jax.numpy as jnp
from jax import lax
from jax.experimental import pallas as pl
from jax.experimental.pallas import tpu as pltpu
from jax.experimental.pallas import tpu_sc as plsc

INPUT_DIM = 768
HIDDEN = INPUT_DIM // 2
NUM_EXPERTS = 64
N_TOKENS = 32768
BT = 2048

NC = 2
NS = 16
NW = NC * NS
TPW = N_TOKENS // NW
L = 16


def _cand_kernel(x_ref, w1_ref, b1_ref, w2_ref, b2_ref, out_ref):
    h = jnp.dot(x_ref[:], w1_ref[:], preferred_element_type=jnp.float32)
    h = jnp.maximum(h + b1_ref[:], 0.0)
    logits = jnp.dot(h, w2_ref[:], preferred_element_type=jnp.float32)
    logits = logits + b2_ref[:]

    lane_f = jax.lax.broadcasted_iota(jnp.int32, logits.shape, 1).astype(
        jnp.float32)
    m1 = jnp.max(logits, axis=-1, keepdims=True)
    i1f = jnp.min(jnp.where(logits == m1, lane_f, float(NUM_EXPERTS)),
                  axis=-1, keepdims=True)
    masked = jnp.where(lane_f == i1f, -jnp.inf, logits)
    m2 = jnp.max(masked, axis=-1, keepdims=True)
    i2f = jnp.min(jnp.where(masked == m2, lane_f, float(NUM_EXPERTS)),
                  axis=-1, keepdims=True)
    out = jnp.concatenate([m1, m2, i1f, i2f], axis=1)
    out_ref[:] = out.T


def _tc_candidates(x, W1, b1, W2, b2):
    n = x.shape[0]
    return pl.pallas_call(
        _cand_kernel,
        grid=(n // BT,),
        in_specs=[
            pl.BlockSpec((BT, INPUT_DIM), lambda i: (i, 0)),
            pl.BlockSpec((INPUT_DIM, HIDDEN), lambda i: (0, 0)),
            pl.BlockSpec((1, HIDDEN), lambda i: (0, 0)),
            pl.BlockSpec((HIDDEN, NUM_EXPERTS), lambda i: (0, 0)),
            pl.BlockSpec((1, NUM_EXPERTS), lambda i: (0, 0)),
        ],
        out_specs=pl.BlockSpec((4, BT), lambda i: (0, i)),
        out_shape=jax.ShapeDtypeStruct((4, n), jnp.float32),
        compiler_params=pltpu.CompilerParams(
            dimension_semantics=("parallel",),
        ),
    )(x, W1, b1.reshape(1, HIDDEN), W2, b2.reshape(1, NUM_EXPERTS))


def _sc_body(cand_hbm, ow_hbm, oi_hbm, vm, ow, oi):
    wid = lax.axis_index("s") * NC + lax.axis_index("c")
    base = wid * TPW
    pltpu.sync_copy(cand_hbm.at[:, pl.ds(base, TPW)], vm)

    def body(t, _):
        tt = t * L
        m1 = vm[0, pl.ds(tt, L)]
        m2 = vm[1, pl.ds(tt, L)]
        i1 = vm[2, pl.ds(tt, L)]
        i2 = vm[3, pl.ds(tt, L)]
        e2 = jnp.exp(m2 - m1)
        inv = 1.0 / (1.0 + e2)
        ow[0, pl.ds(tt, L)] = inv
        ow[1, pl.ds(tt, L)] = e2 * inv
        oi[0, pl.ds(tt, L)] = i1.astype(jnp.int32)
        oi[1, pl.ds(tt, L)] = i2.astype(jnp.int32)
        return 0

    lax.fori_loop(0, TPW // L, body, 0)
    pltpu.sync_copy(ow, ow_hbm.at[:, pl.ds(base, TPW)])
    pltpu.sync_copy(oi, oi_hbm.at[:, pl.ds(base, TPW)])


@functools.cache
def _sc_softmax():
    return pl.kernel(
        _sc_body,
        out_type=[
            jax.ShapeDtypeStruct((2, N_TOKENS), jnp.float32),
            jax.ShapeDtypeStruct((2, N_TOKENS), jnp.int32),
        ],
        mesh=plsc.VectorSubcoreMesh(
            core_axis_name="c", subcore_axis_name="s",
            num_cores=NC, num_subcores=NS),
        scratch_types=[
            pltpu.VMEM((4, TPW), jnp.float32),
            pltpu.VMEM((2, TPW), jnp.float32),
            pltpu.VMEM((2, TPW), jnp.int32),
        ],
    )


@jax.jit
def kernel(x, W1, b1, W2, b2):
    cand = _tc_candidates(x, W1, b1, W2, b2)
    ow, oi = _sc_softmax()(cand)
    return (ow.T, oi.T)
